# baseline (speedup 1.0000x reference)
import functools

import jax
import jax.numpy as jnp
from jax import lax
from jax.experimental import pallas as pl
from jax.experimental.pallas import tpu as pltpu
from jax.experimental.pallas import tpu_sc as plsc

_N = 10000
_E = 320000
_D = 128
_H = 32
_NC = 2
_NS = 16
_NTILES = _NC * _NS
_EC = 128
_REDGE = _E // _EC
_RNOM = 80
_DH = _D // 2
_NSLICE = 632
_NPAD = _NS * _NSLICE
_PADIDX = 10008
_BM = 2000

_HIGH = lax.Precision.HIGHEST


def _fill_pad(idx_v, lo, hi):
    def fill(i, _):
        for l in range(_EC // 16):
            idx_v[i, pl.ds(l * 16, 16)] = jnp.full((16,), _PADIDX, jnp.int32)
        return 0

    lax.fori_loop(lo, hi, fill, 0)


def _sc_degree(e3):
    mesh = plsc.VectorSubcoreMesh(core_axis_name="c", subcore_axis_name="s")
    nbuf = 2

    @functools.partial(
        pl.kernel,
        out_type=jax.ShapeDtypeStruct((_NC * _NPAD,), jnp.float32),
        mesh=mesh,
        scratch_types=[
            pltpu.VMEM((_RNOM, _EC), jnp.int32),
            pltpu.VMEM((_EC,), jnp.float32),
            pltpu.VMEM((640,), jnp.float32),
            pltpu.VMEM_SHARED((_NPAD,), jnp.float32),
            [pltpu.SemaphoreType.DMA for _ in range(nbuf)],
        ],
        compiler_params=pltpu.CompilerParams(use_tc_tiling_on_sc=False),
    )
    def k(e_hbm, out_hbm, idx_v, ones_v, z_v, deg_sh, dsem):
        c = lax.axis_index("c")
        s = lax.axis_index("s")
        wid = s * _NC + c

        def zfill(i, _):
            z_v[pl.ds(i * 16, 16)] = jnp.zeros((16,), jnp.float32)
            return 0

        lax.fori_loop(0, 40, zfill, 0)
        for l in range(_EC // 16):
            ones_v[pl.ds(l * 16, 16)] = jnp.ones((16,), jnp.float32)
        base = s * _NSLICE
        pltpu.sync_copy(z_v.at[pl.ds(0, _NSLICE)],
                        deg_sh.at[pl.ds(base, _NSLICE)])
        plsc.subcore_barrier()

        dst_hbm = e_hbm.at[1]
        row0 = wid * _RNOM

        nreal = _REDGE - (_NTILES - 1) * _RNOM

        @pl.when(wid < _NTILES - 1)
        def _():
            pltpu.sync_copy(dst_hbm.at[pl.ds(row0, _RNOM)], idx_v)

        @pl.when(wid == _NTILES - 1)
        def _():
            pltpu.sync_copy(dst_hbm.at[pl.ds(row0, nreal)],
                            idx_v.at[pl.ds(0, nreal)])
            _fill_pad(idx_v, nreal, _RNOM)

        def scat(j, b):
            pltpu.async_copy(ones_v, deg_sh.at[idx_v.at[j]], dsem[b],
                             add=True)

        def scat_wait(j, b):
            pltpu.make_async_copy(ones_v, deg_sh.at[idx_v.at[j]],
                                  dsem[b]).wait()

        for b in range(nbuf):
            scat(b, b)

        def pair(i, _):
            j0 = i * nbuf
            for b in range(nbuf):
                scat_wait(j0 + b, b)
                scat(j0 + nbuf + b, b)
            return 0

        lax.fori_loop(0, _RNOM // nbuf - 1, pair, 0)
        j0 = _RNOM - nbuf
        for b in range(nbuf):
            scat_wait(j0 + b, b)
        plsc.subcore_barrier()
        pltpu.sync_copy(deg_sh.at[pl.ds(base, _NSLICE)],
                        z_v.at[pl.ds(0, _NSLICE)])
        pltpu.sync_copy(z_v.at[pl.ds(0, _NSLICE)],
                        out_hbm.at[pl.ds(c * _NPAD + base, _NSLICE)])

    return k(e3).reshape(_NC, _NPAD)


def _sc_scatter(y2, e3):
    mesh = plsc.VectorSubcoreMesh(core_axis_name="c", subcore_axis_name="s")

    nbuf = 8
    nphase = 2
    rphase = 160 // nphase
    nquad = rphase // nbuf

    @functools.partial(
        pl.kernel,
        out_type=jax.ShapeDtypeStruct((_NC, _NPAD, _DH), jnp.float32),
        mesh=mesh,
        scratch_types=[
            pltpu.VMEM((rphase, _EC), jnp.int32),
            pltpu.VMEM((rphase, _EC), jnp.int32),
            [pltpu.VMEM((_EC, _DH), jnp.float32) for _ in range(nbuf)],
            pltpu.VMEM_SHARED((_NPAD, _DH), jnp.float32),
            [pltpu.SemaphoreType.DMA for _ in range(nbuf)],
            [pltpu.SemaphoreType.DMA for _ in range(nbuf)],
        ],
        compiler_params=pltpu.CompilerParams(use_tc_tiling_on_sc=False),
    )
    def k(y_hbm, e_hbm, out_hbm, src_v, dst_v, rows, agg_sh, gsem, ssem):
        c = lax.axis_index("c")
        s = lax.axis_index("s")

        def zrow(i, _):
            for l in range(_DH // 16):
                rows[0][i, pl.ds(l * 16, 16)] = jnp.zeros((16,), jnp.float32)
            return 0

        lax.fori_loop(0, _EC, zrow, 0)
        base = s * _NSLICE
        for q in range(_NSLICE // _EC):
            pltpu.sync_copy(rows[0], agg_sh.at[pl.ds(base + q * _EC, _EC)])
        rem = _NSLICE % _EC
        pltpu.sync_copy(rows[0].at[pl.ds(0, rem)],
                        agg_sh.at[pl.ds(base + _NSLICE - rem, rem)])
        plsc.subcore_barrier()

        ytab = y_hbm.at[c]
        src_hbm = e_hbm.at[0]
        dst_hbm = e_hbm.at[1]

        def gather(j, b):
            pltpu.async_copy(ytab.at[src_v.at[j]], rows[b], gsem[b])

        def gather_wait(j, b):
            pltpu.make_async_copy(ytab.at[src_v.at[j]], rows[b],
                                  gsem[b]).wait()

        def scatter(j, b):
            pltpu.async_copy(rows[b], agg_sh.at[dst_v.at[j]], ssem[b],
                             add=True)

        def scatter_wait(j, b):
            pltpu.make_async_copy(rows[b], agg_sh.at[dst_v.at[j]],
                                  ssem[b]).wait()

        for p in range(nphase):
            row0 = s * 160 + p * rphase
            tail0 = (_NS - 1) * 160 + p * rphase
            if tail0 + rphase <= _REDGE:
                pltpu.sync_copy(src_hbm.at[pl.ds(row0, rphase)], src_v)
                pltpu.sync_copy(dst_hbm.at[pl.ds(row0, rphase)], dst_v)
            else:
                nreal = _REDGE - tail0

                @pl.when(s < _NS - 1)
                def _():
                    pltpu.sync_copy(src_hbm.at[pl.ds(row0, rphase)], src_v)
                    pltpu.sync_copy(dst_hbm.at[pl.ds(row0, rphase)], dst_v)

                @pl.when(s == _NS - 1)
                def _():
                    pltpu.sync_copy(src_hbm.at[pl.ds(row0, nreal)],
                                    src_v.at[pl.ds(0, nreal)])
                    pltpu.sync_copy(dst_hbm.at[pl.ds(row0, nreal)],
                                    dst_v.at[pl.ds(0, nreal)])
                    _fill_pad(src_v, nreal, rphase)
                    _fill_pad(dst_v, nreal, rphase)

            for b in range(nbuf):
                gather(b, b)

            def quad(i, _):
                j0 = i * nbuf
                for b in range(nbuf):
                    gather_wait(j0 + b, b)
                    scatter(j0 + b, b)
                for b in range(nbuf):
                    scatter_wait(j0 + b, b)
                    gather(j0 + nbuf + b, b)
                return 0

            lax.fori_loop(0, nquad - 1, quad, 0)
            j0 = (nquad - 1) * nbuf
            for b in range(nbuf):
                gather_wait(j0 + b, b)
                scatter(j0 + b, b)
            for b in range(nbuf):
                scatter_wait(j0 + b, b)
        plsc.subcore_barrier()
        for q in range(_NSLICE // _EC):
            b = q % nbuf
            pltpu.sync_copy(agg_sh.at[pl.ds(base + q * _EC, _EC)], rows[b])
            pltpu.sync_copy(rows[b],
                            out_hbm.at[c, pl.ds(base + q * _EC, _EC)])
        pltpu.sync_copy(agg_sh.at[pl.ds(base + _NSLICE - rem, rem)],
                        rows[0].at[pl.ds(0, rem)])
        pltpu.sync_copy(rows[0].at[pl.ds(0, rem)],
                        out_hbm.at[c, pl.ds(base + _NSLICE - rem, rem)])

    return k(y2, e3)


def _tc_prep(x, Wg, degT):
    def body(x_ref, wg_ref, deg_ref, y_ref):
        deg = deg_ref[:, 0:1] + deg_ref[:, 1:2] + 1.0
        dinv = lax.rsqrt(deg)
        xw = jnp.dot(x_ref[...], wg_ref[...], precision=_HIGH,
                     preferred_element_type=jnp.float32)
        y = xw * dinv
        y_ref[0] = y[:, :_DH]
        y_ref[1] = y[:, _DH:]

    return pl.pallas_call(
        body,
        grid=(_N // _BM,),
        in_specs=[
            pl.BlockSpec((_BM, _D), lambda i: (i, 0)),
            pl.BlockSpec((_D, _D), lambda i: (0, 0)),
            pl.BlockSpec((_BM, 2), lambda i: (i, 0)),
        ],
        out_specs=pl.BlockSpec((_NC, _BM, _DH), lambda i: (0, i, 0)),
        out_shape=jax.ShapeDtypeStruct((_NC, _NPAD, _DH), jnp.float32),
    )(x, Wg, degT)


def _tc_head(agg, y2, degT, x, act2, price2, bg2, w1a, w1t, b12, W2, b22,
             W3, b32):
    grid_n = _N // _BM

    def body(agg_ref, y_ref, deg_ref, x_ref, act_ref, price_ref, bg_ref,
             w1a_ref, w1t_ref, b1_ref, w2_ref, b2_ref, w3_ref, b3_ref,
             o_ref, acc_ref):
        i = pl.program_id(0)
        deg = deg_ref[:, 0:1] + deg_ref[:, 1:2] + 1.0
        dinv = lax.rsqrt(deg)
        aggsum = (jnp.concatenate([agg_ref[0], agg_ref[1]], axis=1)
                  + jnp.concatenate([y_ref[0], y_ref[1]], axis=1))
        out_pre = aggsum * dinv + bg_ref[...]
        h = jnp.maximum(out_pre, 0.0) + x_ref[...]
        z1 = (jnp.dot(h, w1a_ref[...], precision=_HIGH,
                      preferred_element_type=jnp.float32)
              + act_ref[...] * w1t_ref[0:1, :]
              + price_ref[...] * w1t_ref[1:2, :]
              + b1_ref[...])
        z1 = jnp.maximum(z1, 0.0)
        z2 = jnp.dot(z1, w2_ref[...], precision=_HIGH,
                     preferred_element_type=jnp.float32) + b2_ref[...]
        z2 = jnp.maximum(z2, 0.0)
        part = jnp.sum(z2, axis=0, keepdims=True)

        @pl.when(i == 0)
        def _():
            acc_ref[...] = part

        @pl.when(i > 0)
        def _():
            acc_ref[...] = acc_ref[...] + part

        @pl.when(i == grid_n - 1)
        def _():
            o_ref[...] = jnp.dot(acc_ref[...], w3_ref[...], precision=_HIGH,
                                 preferred_element_type=jnp.float32) + b3_ref[...]

    return pl.pallas_call(
        body,
        grid=(grid_n,),
        in_specs=[
            pl.BlockSpec((_NC, _BM, _DH), lambda i: (0, i, 0)),
            pl.BlockSpec((_NC, _BM, _DH), lambda i: (0, i, 0)),
            pl.BlockSpec((_BM, 2), lambda i: (i, 0)),
            pl.BlockSpec((_BM, _D), lambda i: (i, 0)),
            pl.BlockSpec((_BM, 1), lambda i: (i, 0)),
            pl.BlockSpec((1, 1), lambda i: (0, 0)),
            pl.BlockSpec((1, _D), lambda i: (0, 0)),
            pl.BlockSpec((_D, _H), lambda i: (0, 0)),
            pl.BlockSpec((2, _H), lambda i: (0, 0)),
            pl.BlockSpec((1, _H), lambda i: (0, 0)),
            pl.BlockSpec((_H, _H), lambda i: (0, 0)),
            pl.BlockSpec((1, _H), lambda i: (0, 0)),
            pl.BlockSpec((_H, 1), lambda i: (0, 0)),
            pl.BlockSpec((1, 1), lambda i: (0, 0)),
        ],
        out_specs=pl.BlockSpec((1, 1), lambda i: (0, 0)),
        out_shape=jax.ShapeDtypeStruct((1, 1), jnp.float32),
        scratch_shapes=[pltpu.VMEM((1, _H), jnp.float32)],
    )(agg, y2, degT, x, act2, price2, bg2, w1a, w1t, b12, W2, b22, W3, b32)


def kernel(x, edge_index, action, price, Wg, bg, W1, b1, W2, b2, W3, b3):
    e3 = edge_index.reshape(2, _REDGE, _EC)

    degp = _sc_degree(e3)
    degT = degp.T
    y2 = _tc_prep(x, Wg, degT)
    agg = _sc_scatter(y2, e3)

    v2 = _tc_head(
        agg, y2, degT, x,
        action[:, None],
        price.reshape(1, 1),
        bg[None, :],
        W1[:_D],
        W1[_D:],
        b1[None, :],
        W2,
        b2[None, :],
        W3,
        b3[None, :],
    )
    return v2[0, 0]


# speedup vs baseline: 1.2114x; 1.2114x over previous
import functools

import jax
import jax.numpy as jnp
from jax import lax
from jax.experimental import pallas as pl
from jax.experimental.pallas import tpu as pltpu
from jax.experimental.pallas import tpu_sc as plsc

_N = 10000
_E = 320000
_D = 128
_H = 32
_NC = 2
_NS = 16
_NTILES = _NC * _NS
_EC = 128
_REDGE = _E // _EC
_RNOM = 80
_DH = _D // 2
_NSLICE = 632
_NPAD = _NS * _NSLICE
_PADIDX = 10008
_BM = 2000

_HIGH = lax.Precision.HIGHEST


def _fill_pad(idx_v, lo, hi):
    def fill(i, _):
        for l in range(_EC // 16):
            idx_v[i, pl.ds(l * 16, 16)] = jnp.full((16,), _PADIDX, jnp.int32)
        return 0

    lax.fori_loop(lo, hi, fill, 0)


def _sc_degree(e3):
    mesh = plsc.VectorSubcoreMesh(core_axis_name="c", subcore_axis_name="s")
    nbuf = 2

    @functools.partial(
        pl.kernel,
        out_type=jax.ShapeDtypeStruct((_NC * _NPAD,), jnp.float32),
        mesh=mesh,
        scratch_types=[
            pltpu.VMEM((_RNOM, _EC), jnp.int32),
            pltpu.VMEM((_EC,), jnp.float32),
            pltpu.VMEM((640,), jnp.float32),
            pltpu.VMEM_SHARED((_NPAD,), jnp.float32),
            [pltpu.SemaphoreType.DMA for _ in range(nbuf)],
        ],
        compiler_params=pltpu.CompilerParams(use_tc_tiling_on_sc=False),
    )
    def k(e_hbm, out_hbm, idx_v, ones_v, z_v, deg_sh, dsem):
        c = lax.axis_index("c")
        s = lax.axis_index("s")
        wid = s * _NC + c

        def zfill(i, _):
            z_v[pl.ds(i * 16, 16)] = jnp.zeros((16,), jnp.float32)
            return 0

        lax.fori_loop(0, 40, zfill, 0)
        for l in range(_EC // 16):
            ones_v[pl.ds(l * 16, 16)] = jnp.ones((16,), jnp.float32)
        base = s * _NSLICE
        pltpu.sync_copy(z_v.at[pl.ds(0, _NSLICE)],
                        deg_sh.at[pl.ds(base, _NSLICE)])
        plsc.subcore_barrier()

        dst_hbm = e_hbm.at[1]
        row0 = wid * _RNOM

        nreal = _REDGE - (_NTILES - 1) * _RNOM

        @pl.when(wid < _NTILES - 1)
        def _():
            pltpu.sync_copy(dst_hbm.at[pl.ds(row0, _RNOM)], idx_v)

        @pl.when(wid == _NTILES - 1)
        def _():
            pltpu.sync_copy(dst_hbm.at[pl.ds(row0, nreal)],
                            idx_v.at[pl.ds(0, nreal)])
            _fill_pad(idx_v, nreal, _RNOM)

        def scat(j, b):
            pltpu.async_copy(ones_v, deg_sh.at[idx_v.at[j]], dsem[b],
                             add=True)

        def scat_wait(j, b):
            pltpu.make_async_copy(ones_v, deg_sh.at[idx_v.at[j]],
                                  dsem[b]).wait()

        for b in range(nbuf):
            scat(b, b)

        def pair(i, _):
            j0 = i * nbuf
            for b in range(nbuf):
                scat_wait(j0 + b, b)
                scat(j0 + nbuf + b, b)
            return 0

        lax.fori_loop(0, _RNOM // nbuf - 1, pair, 0)
        j0 = _RNOM - nbuf
        for b in range(nbuf):
            scat_wait(j0 + b, b)
        plsc.subcore_barrier()
        pltpu.sync_copy(deg_sh.at[pl.ds(base, _NSLICE)],
                        z_v.at[pl.ds(0, _NSLICE)])
        pltpu.sync_copy(z_v.at[pl.ds(0, _NSLICE)],
                        out_hbm.at[pl.ds(c * _NPAD + base, _NSLICE)])

    return k(e3).reshape(_NC, _NPAD)


def _sc_scatter(y2, e3):
    mesh = plsc.VectorSubcoreMesh(core_axis_name="c", subcore_axis_name="s")

    nbuf = 8
    nphase = 2
    rphase = _RNOM // nphase
    nquad = rphase // nbuf

    @functools.partial(
        pl.kernel,
        out_type=jax.ShapeDtypeStruct((_NC, _NPAD, _D), jnp.bfloat16),
        mesh=mesh,
        scratch_types=[
            pltpu.VMEM((rphase, _EC), jnp.int32),
            pltpu.VMEM((rphase, _EC), jnp.int32),
            [pltpu.VMEM((_EC, _D), jnp.bfloat16) for _ in range(nbuf)],
            pltpu.VMEM_SHARED((_NPAD, _D), jnp.bfloat16),
            [pltpu.SemaphoreType.DMA for _ in range(nbuf)],
            [pltpu.SemaphoreType.DMA for _ in range(nbuf)],
        ],
        compiler_params=pltpu.CompilerParams(use_tc_tiling_on_sc=False),
    )
    def k(y_hbm, e_hbm, out_hbm, src_v, dst_v, rows, agg_sh, gsem, ssem):
        c = lax.axis_index("c")
        s = lax.axis_index("s")
        wid = s * _NC + c

        def zrow(i, _):
            for l in range(_D // 32):
                rows[0][i, pl.ds(l * 32, 32)] = jnp.zeros((32,),
                                                          jnp.bfloat16)
            return 0

        lax.fori_loop(0, _EC, zrow, 0)
        base = s * _NSLICE
        for q in range(_NSLICE // _EC):
            pltpu.sync_copy(rows[0], agg_sh.at[pl.ds(base + q * _EC, _EC)])
        rem = _NSLICE % _EC
        pltpu.sync_copy(rows[0].at[pl.ds(0, rem)],
                        agg_sh.at[pl.ds(base + _NSLICE - rem, rem)])
        plsc.subcore_barrier()

        ytab = y_hbm
        src_hbm = e_hbm.at[0]
        dst_hbm = e_hbm.at[1]

        def gather(j, b):
            pltpu.async_copy(ytab.at[src_v.at[j]], rows[b], gsem[b])

        def gather_wait(j, b):
            pltpu.make_async_copy(ytab.at[src_v.at[j]], rows[b],
                                  gsem[b]).wait()

        def scatter(j, b):
            pltpu.async_copy(rows[b], agg_sh.at[dst_v.at[j]], ssem[b],
                             add=True)

        def scatter_wait(j, b):
            pltpu.make_async_copy(rows[b], agg_sh.at[dst_v.at[j]],
                                  ssem[b]).wait()

        for p in range(nphase):
            row0 = wid * _RNOM + p * rphase
            tail0 = (_NTILES - 1) * _RNOM + p * rphase
            if tail0 + rphase <= _REDGE:
                pltpu.sync_copy(src_hbm.at[pl.ds(row0, rphase)], src_v)
                pltpu.sync_copy(dst_hbm.at[pl.ds(row0, rphase)], dst_v)
            elif tail0 < _REDGE:
                nreal = _REDGE - tail0

                @pl.when(wid < _NTILES - 1)
                def _():
                    pltpu.sync_copy(src_hbm.at[pl.ds(row0, rphase)], src_v)
                    pltpu.sync_copy(dst_hbm.at[pl.ds(row0, rphase)], dst_v)

                @pl.when(wid == _NTILES - 1)
                def _():
                    pltpu.sync_copy(src_hbm.at[pl.ds(row0, nreal)],
                                    src_v.at[pl.ds(0, nreal)])
                    pltpu.sync_copy(dst_hbm.at[pl.ds(row0, nreal)],
                                    dst_v.at[pl.ds(0, nreal)])
                    _fill_pad(src_v, nreal, rphase)
                    _fill_pad(dst_v, nreal, rphase)
            else:

                @pl.when(wid < _NTILES - 1)
                def _():
                    pltpu.sync_copy(src_hbm.at[pl.ds(row0, rphase)], src_v)
                    pltpu.sync_copy(dst_hbm.at[pl.ds(row0, rphase)], dst_v)

                @pl.when(wid == _NTILES - 1)
                def _():
                    _fill_pad(src_v, 0, rphase)
                    _fill_pad(dst_v, 0, rphase)

            for b in range(nbuf):
                gather(b, b)

            def quad(i, _):
                j0 = i * nbuf
                for b in range(nbuf):
                    gather_wait(j0 + b, b)
                    scatter(j0 + b, b)
                for b in range(nbuf):
                    scatter_wait(j0 + b, b)
                    gather(j0 + nbuf + b, b)
                return 0

            lax.fori_loop(0, nquad - 1, quad, 0)
            j0 = (nquad - 1) * nbuf
            for b in range(nbuf):
                gather_wait(j0 + b, b)
                scatter(j0 + b, b)
            for b in range(nbuf):
                scatter_wait(j0 + b, b)
        plsc.subcore_barrier()
        for q in range(_NSLICE // _EC):
            b = q % nbuf
            pltpu.sync_copy(agg_sh.at[pl.ds(base + q * _EC, _EC)], rows[b])
            pltpu.sync_copy(rows[b],
                            out_hbm.at[c, pl.ds(base + q * _EC, _EC)])
        pltpu.sync_copy(agg_sh.at[pl.ds(base + _NSLICE - rem, rem)],
                        rows[0].at[pl.ds(0, rem)])
        pltpu.sync_copy(rows[0].at[pl.ds(0, rem)],
                        out_hbm.at[c, pl.ds(base + _NSLICE - rem, rem)])

    return k(y2, e3)


def _tc_prep(x, Wg, degT):
    def body(x_ref, wg_ref, deg_ref, y_ref):
        deg = deg_ref[:, 0:1] + deg_ref[:, 1:2] + 1.0
        dinv = lax.rsqrt(deg)
        xw = jnp.dot(x_ref[...], wg_ref[...], precision=_HIGH,
                     preferred_element_type=jnp.float32)
        y_ref[...] = (xw * dinv).astype(jnp.bfloat16)

    return pl.pallas_call(
        body,
        grid=(_N // _BM,),
        in_specs=[
            pl.BlockSpec((_BM, _D), lambda i: (i, 0)),
            pl.BlockSpec((_D, _D), lambda i: (0, 0)),
            pl.BlockSpec((_BM, 2), lambda i: (i, 0)),
        ],
        out_specs=pl.BlockSpec((_BM, _D), lambda i: (i, 0)),
        out_shape=jax.ShapeDtypeStruct((_NPAD, _D), jnp.bfloat16),
    )(x, Wg, degT)


def _tc_head(agg, y2, degT, x, act2, price2, bg2, w1a, w1t, b12, W2, b22,
             W3, b32):
    grid_n = _N // _BM

    def body(agg_ref, y_ref, deg_ref, x_ref, act_ref, price_ref, bg_ref,
             w1a_ref, w1t_ref, b1_ref, w2_ref, b2_ref, w3_ref, b3_ref,
             o_ref, acc_ref):
        i = pl.program_id(0)
        deg = deg_ref[:, 0:1] + deg_ref[:, 1:2] + 1.0
        dinv = lax.rsqrt(deg)
        aggsum = (agg_ref[0].astype(jnp.float32)
                  + agg_ref[1].astype(jnp.float32)
                  + y_ref[...].astype(jnp.float32))
        out_pre = aggsum * dinv + bg_ref[...]
        h = jnp.maximum(out_pre, 0.0) + x_ref[...]
        z1 = (jnp.dot(h, w1a_ref[...], precision=_HIGH,
                      preferred_element_type=jnp.float32)
              + act_ref[...] * w1t_ref[0:1, :]
              + price_ref[...] * w1t_ref[1:2, :]
              + b1_ref[...])
        z1 = jnp.maximum(z1, 0.0)
        z2 = jnp.dot(z1, w2_ref[...], precision=_HIGH,
                     preferred_element_type=jnp.float32) + b2_ref[...]
        z2 = jnp.maximum(z2, 0.0)
        part = jnp.sum(z2, axis=0, keepdims=True)

        @pl.when(i == 0)
        def _():
            acc_ref[...] = part

        @pl.when(i > 0)
        def _():
            acc_ref[...] = acc_ref[...] + part

        @pl.when(i == grid_n - 1)
        def _():
            o_ref[...] = jnp.dot(acc_ref[...], w3_ref[...], precision=_HIGH,
                                 preferred_element_type=jnp.float32) + b3_ref[...]

    return pl.pallas_call(
        body,
        grid=(grid_n,),
        in_specs=[
            pl.BlockSpec((_NC, _BM, _D), lambda i: (0, i, 0)),
            pl.BlockSpec((_BM, _D), lambda i: (i, 0)),
            pl.BlockSpec((_BM, 2), lambda i: (i, 0)),
            pl.BlockSpec((_BM, _D), lambda i: (i, 0)),
            pl.BlockSpec((_BM, 1), lambda i: (i, 0)),
            pl.BlockSpec((1, 1), lambda i: (0, 0)),
            pl.BlockSpec((1, _D), lambda i: (0, 0)),
            pl.BlockSpec((_D, _H), lambda i: (0, 0)),
            pl.BlockSpec((2, _H), lambda i: (0, 0)),
            pl.BlockSpec((1, _H), lambda i: (0, 0)),
            pl.BlockSpec((_H, _H), lambda i: (0, 0)),
            pl.BlockSpec((1, _H), lambda i: (0, 0)),
            pl.BlockSpec((_H, 1), lambda i: (0, 0)),
            pl.BlockSpec((1, 1), lambda i: (0, 0)),
        ],
        out_specs=pl.BlockSpec((1, 1), lambda i: (0, 0)),
        out_shape=jax.ShapeDtypeStruct((1, 1), jnp.float32),
        scratch_shapes=[pltpu.VMEM((1, _H), jnp.float32)],
    )(agg, y2, degT, x, act2, price2, bg2, w1a, w1t, b12, W2, b22, W3, b32)


def kernel(x, edge_index, action, price, Wg, bg, W1, b1, W2, b2, W3, b3):
    e3 = edge_index.reshape(2, _REDGE, _EC)

    degp = _sc_degree(e3)
    degT = degp.T
    y2 = _tc_prep(x, Wg, degT)
    agg = _sc_scatter(y2, e3)

    v2 = _tc_head(
        agg, y2, degT, x,
        action[:, None],
        price.reshape(1, 1),
        bg[None, :],
        W1[:_D],
        W1[_D:],
        b1[None, :],
        W2,
        b2[None, :],
        W3,
        b3[None, :],
    )
    return v2[0, 0]
